# baseline (speedup 1.0000x reference)
import jax
import jax.numpy as jnp
from jax.experimental import pallas as pl
from jax.experimental.pallas import tpu as pltpu

B, S, D = 2, 2048, 768
E, K = 8, 2
GS = 1024
MLP = 3072
CAP = (GS * K) // E
G = (B * S) // GS


def _router_body(x_ref, wr_ref, cols_ref, aux_ref):
    wr = wr_ref[...]
    iota_e = jax.lax.broadcasted_iota(jnp.int32, (GS, E), 1).astype(jnp.float32)
    tri =(jax.lax.broadcasted_iota(jnp.int32, (GS, GS), 1)
           < jax.lax.broadcasted_iota(jnp.int32, (GS, GS), 0)
           ).astype(jnp.float32)
    imps = []
    for g in range(G):
        xg = x_ref[g]
        logits = jax.lax.dot_general(
            xg, wr, (((1,), (0,)), ((), ())),
            preferred_element_type=jnp.float32)
        gates = jax.nn.softmax(logits, axis=-1)
        m1 = jnp.max(gates, axis=-1, keepdims=True)
        i1 = jnp.min(jnp.where(gates == m1, iota_e, float(E)), axis=-1,
                     keepdims=True)
        oh1 = (iota_e == i1).astype(jnp.float32)
        gmask = gates - oh1 * 1e30
        m2 = jnp.max(gmask, axis=-1, keepdims=True)
        i2 = jnp.min(jnp.where(gmask == m2, iota_e, float(E)), axis=-1,
                     keepdims=True)
        oh2 = (iota_e == i2).astype(jnp.float32)
        exc = jax.lax.dot_general(
            tri, jnp.concatenate([oh1, oh2], axis=1),
            (((1,), (0,)), ((), ())), preferred_element_type=jnp.float32)
        tot0 = jnp.sum(oh1, axis=0, keepdims=True)
        p0 = jnp.sum(oh1 * exc[:, :E], axis=-1, keepdims=True)
        p1 = jnp.sum(oh2 * (exc[:, E:] + tot0), axis=-1, keepdims=True)
        k0 = (p0 < CAP).astype(jnp.float32)
        k1 = (p1 < CAP).astype(jnp.float32)
        gate0 = jnp.sum(oh1 * gates, axis=-1, keepdims=True)
        gate1 = jnp.sum(oh2 * gates, axis=-1, keepdims=True)
        rows = pl.ds(g * GS, GS)
        cols_ref[rows, :] = jnp.concatenate([
            i1, jnp.where(k0 > 0, p0, -1.0), gate0 * k0,
            i2, jnp.where(k1 > 0, p1, -1.0), gate1 * k1,
            jnp.zeros((GS, 2), jnp.float32)], axis=1)
        imps.append(jnp.sum(gates, axis=0, keepdims=True))
    imp = jnp.concatenate(imps, axis=0)
    mean = jnp.mean(imp, axis=-1, keepdims=True)
    var = jnp.mean((imp - mean) ** 2, axis=-1, keepdims=True)
    aux = jnp.mean(var / (mean + 1e-10) ** 2)
    aux_ref[...] = jnp.full((1, 1), aux, jnp.float32)


def _moe_body(x_ref, cols_ref, w1_ref, b1_ref, w2_ref, b2_ref, out_ref,
              xe_ref):
    e = pl.program_id(0)
    m = pl.program_id(1)
    ef = e.astype(jnp.float32)
    iota_c = jax.lax.broadcasted_iota(jnp.int32, (GS, CAP), 1).astype(jnp.float32)
    w1 = w1_ref[0]
    w2 = w2_ref[0]
    b1 = b1_ref[0]
    b2 = b2_ref[0]

    @pl.when((e == 0) & (m == 0))
    def _():
        out_ref[...] = jnp.zeros_like(out_ref)

    for g in range(G):
        cols = cols_ref[pl.ds(g * GS, GS), :]
        me0 = cols[:, 0:1] == ef
        me1 = cols[:, 3:4] == ef
        d0 = (me0 & (cols[:, 1:2] == iota_c)).astype(jnp.float32)
        d1 = (me1 & (cols[:, 4:5] == iota_c)).astype(jnp.float32)
        comb = d0 * cols[:, 2:3] + d1 * cols[:, 5:6]

        @pl.when(m == 0)
        def _():
            disp = d0 + d1
            xe_ref[g] = jax.lax.dot_general(
                disp, x_ref[g], (((0,), (0,)), ((), ())),
                preferred_element_type=jnp.float32)
            out_ref[g] += jnp.sum(comb, axis=1, keepdims=True) * b2

        h = jax.lax.dot_general(
            xe_ref[g], w1, (((1,), (0,)), ((), ())),
            preferred_element_type=jnp.float32) + b1
        h = jax.nn.gelu(h)
        t = jax.lax.dot_general(
            h, w2, (((1,), (0,)), ((), ())),
            preferred_element_type=jnp.float32)
        out_ref[g] += jax.lax.dot_general(
            comb, t, (((1,), (0,)), ((), ())),
            preferred_element_type=jnp.float32)


def kernel(inputs, w_router, w1, b1, w2, b2):
    x = inputs.reshape(G, GS, D)
    rf = jnp.float32
    router_out = pl.pallas_call(
        _router_body,
        out_shape=[
            jax.ShapeDtypeStruct((G * GS, 8), rf),
            jax.ShapeDtypeStruct((1, 1), rf),
        ],
    )(x, w_router)
    cols, aux = router_out

    MC = 4
    MB = MLP // MC
    out = pl.pallas_call(
        _moe_body,
        grid=(E, MC),
        in_specs=[
            pl.BlockSpec((G, GS, D), lambda e, m: (0, 0, 0)),
            pl.BlockSpec((G * GS, 8), lambda e, m: (0, 0)),
            pl.BlockSpec((1, D, MB), lambda e, m: (e, 0, m)),
            pl.BlockSpec((1, 1, MB), lambda e, m: (e, 0, m)),
            pl.BlockSpec((1, MB, D), lambda e, m: (e, m, 0)),
            pl.BlockSpec((1, 1, D), lambda e, m: (e, 0, 0)),
        ],
        out_specs=pl.BlockSpec((G, GS, D), lambda e, m: (0, 0, 0)),
        out_shape=jax.ShapeDtypeStruct((G, GS, D), jnp.float32),
        scratch_shapes=[pltpu.VMEM((G, CAP, D), jnp.float32)],
        compiler_params=pltpu.CompilerParams(
            dimension_semantics=("arbitrary", "arbitrary")),
    )(x, cols, w1, b1.reshape(E, 1, MLP), w2, b2.reshape(E, 1, D))

    out = out.reshape(B, S, D)
    aux = aux.reshape(())
    return out, {"auxiliary_loss": aux, "importance_loss": aux}


# speedup vs baseline: 1.0726x; 1.0726x over previous
import jax
import jax.numpy as jnp
from jax.experimental import pallas as pl
from jax.experimental.pallas import tpu as pltpu

B, S, D = 2, 2048, 768
E, K = 8, 2
GS = 1024
MLP = 3072
CAP = (GS * K) // E
G = (B * S) // GS


def _router_body(x_ref, wr_ref, cols_ref, aux_ref):
    wr = wr_ref[...]
    iota_e = jax.lax.broadcasted_iota(jnp.int32, (GS, E), 1).astype(jnp.float32)
    tri =(jax.lax.broadcasted_iota(jnp.int32, (GS, GS), 1)
           < jax.lax.broadcasted_iota(jnp.int32, (GS, GS), 0)
           ).astype(jnp.float32)
    imps = []
    for g in range(G):
        xg = x_ref[g]
        logits = jax.lax.dot_general(
            xg, wr, (((1,), (0,)), ((), ())),
            preferred_element_type=jnp.float32)
        gates = jax.nn.softmax(logits, axis=-1)
        m1 = jnp.max(gates, axis=-1, keepdims=True)
        i1 = jnp.min(jnp.where(gates == m1, iota_e, float(E)), axis=-1,
                     keepdims=True)
        oh1 = (iota_e == i1).astype(jnp.float32)
        gmask = gates - oh1 * 1e30
        m2 = jnp.max(gmask, axis=-1, keepdims=True)
        i2 = jnp.min(jnp.where(gmask == m2, iota_e, float(E)), axis=-1,
                     keepdims=True)
        oh2 = (iota_e == i2).astype(jnp.float32)
        exc = jax.lax.dot_general(
            tri, jnp.concatenate([oh1, oh2], axis=1),
            (((1,), (0,)), ((), ())), preferred_element_type=jnp.float32)
        tot0 = jnp.sum(oh1, axis=0, keepdims=True)
        p0 = jnp.sum(oh1 * exc[:, :E], axis=-1, keepdims=True)
        p1 = jnp.sum(oh2 * (exc[:, E:] + tot0), axis=-1, keepdims=True)
        k0 = (p0 < CAP).astype(jnp.float32)
        k1 = (p1 < CAP).astype(jnp.float32)
        gate0 = jnp.sum(oh1 * gates, axis=-1, keepdims=True)
        gate1 = jnp.sum(oh2 * gates, axis=-1, keepdims=True)
        rows = pl.ds(g * GS, GS)
        cols_ref[rows, :] = jnp.concatenate([
            i1, jnp.where(k0 > 0, p0, -1.0), gate0 * k0,
            i2, jnp.where(k1 > 0, p1, -1.0), gate1 * k1,
            jnp.zeros((GS, 2), jnp.float32)], axis=1)
        imps.append(jnp.sum(gates, axis=0, keepdims=True))
    imp = jnp.concatenate(imps, axis=0)
    mean = jnp.mean(imp, axis=-1, keepdims=True)
    var = jnp.mean((imp - mean) ** 2, axis=-1, keepdims=True)
    aux = jnp.mean(var / (mean + 1e-10) ** 2)
    aux_ref[...] = jnp.full((1, 1), aux, jnp.float32)


def _moe_body(x_ref, cols_ref, w1_ref, b1_ref, w2_ref, b2_ref, out_ref,
              xe_ref, comb_ref):
    e = pl.program_id(0)
    m = pl.program_id(1)
    ef = e.astype(jnp.float32)
    iota_c = jax.lax.broadcasted_iota(jnp.int32, (GS, CAP), 1).astype(jnp.float32)
    w1 = w1_ref[0]
    w2 = w2_ref[0]
    b1 = b1_ref[0]
    b2 = b2_ref[0]

    @pl.when((e == 0) & (m == 0))
    def _():
        out_ref[...] = jnp.zeros_like(out_ref)

    for g in range(G):
        @pl.when(m == 0)
        def _():
            cols = cols_ref[pl.ds(g * GS, GS), :]
            me0 = cols[:, 0:1] == ef
            me1 = cols[:, 3:4] == ef
            d0 = me0 & (cols[:, 1:2] == iota_c)
            d1 = me1 & (cols[:, 4:5] == iota_c)
            comb = (d0.astype(jnp.float32) * cols[:, 2:3]
                    + d1.astype(jnp.float32) * cols[:, 5:6])
            comb_ref[g] = comb
            disp = (d0 | d1).astype(jnp.bfloat16)
            xe_ref[g] = jax.lax.dot_general(
                disp, x_ref[g], (((0,), (0,)), ((), ())),
                preferred_element_type=jnp.float32,
            ).astype(jnp.bfloat16)
            out_ref[g] += jnp.sum(comb, axis=1, keepdims=True) * b2

        h = jax.lax.dot_general(
            xe_ref[g], w1, (((1,), (0,)), ((), ())),
            preferred_element_type=jnp.float32) + b1
        h = jax.nn.gelu(h).astype(jnp.bfloat16)
        t = jax.lax.dot_general(
            h, w2, (((1,), (0,)), ((), ())),
            preferred_element_type=jnp.float32)
        out_ref[g] += jax.lax.dot_general(
            comb_ref[g], t, (((1,), (0,)), ((), ())),
            preferred_element_type=jnp.float32)


def kernel(inputs, w_router, w1, b1, w2, b2):
    x = inputs.reshape(G, GS, D)
    rf = jnp.float32
    router_out = pl.pallas_call(
        _router_body,
        out_shape=[
            jax.ShapeDtypeStruct((G * GS, 8), rf),
            jax.ShapeDtypeStruct((1, 1), rf),
        ],
    )(x, w_router)
    cols, aux = router_out

    MC = 2
    MB = MLP // MC
    out = pl.pallas_call(
        _moe_body,
        grid=(E, MC),
        in_specs=[
            pl.BlockSpec((G, GS, D), lambda e, m: (0, 0, 0)),
            pl.BlockSpec((G * GS, 8), lambda e, m: (0, 0)),
            pl.BlockSpec((1, D, MB), lambda e, m: (e, 0, m)),
            pl.BlockSpec((1, 1, MB), lambda e, m: (e, 0, m)),
            pl.BlockSpec((1, MB, D), lambda e, m: (e, m, 0)),
            pl.BlockSpec((1, 1, D), lambda e, m: (e, 0, 0)),
        ],
        out_specs=pl.BlockSpec((G, GS, D), lambda e, m: (0, 0, 0)),
        out_shape=jax.ShapeDtypeStruct((G, GS, D), jnp.float32),
        scratch_shapes=[pltpu.VMEM((G, CAP, D), jnp.bfloat16),
                        pltpu.VMEM((G, GS, CAP), jnp.float32)],
        compiler_params=pltpu.CompilerParams(
            dimension_semantics=("arbitrary", "arbitrary")),
    )(x.astype(jnp.bfloat16), cols, w1.astype(jnp.bfloat16),
      b1.reshape(E, 1, MLP), w2.astype(jnp.bfloat16), b2.reshape(E, 1, D))

    out = out.reshape(B, S, D)
    aux = aux.reshape(())
    return out, {"auxiliary_loss": aux, "importance_loss": aux}
